# baseline (speedup 1.0000x reference)
import functools

import jax
import jax.numpy as jnp
from jax import lax
from jax.experimental import pallas as pl
from jax.experimental.pallas import tpu as pltpu
from jax.experimental.pallas import tpu_sc as plsc

_B = 16384
_F = 16

_info = plsc.get_sparse_core_info()
_NC, _NS = _info.num_cores, _info.num_subcores
_NW = _NC * _NS
_BPW = _B // _NW

_mesh = plsc.VectorSubcoreMesh(core_axis_name="c", subcore_axis_name="s")


@functools.partial(
    pl.kernel,
    out_type=jax.ShapeDtypeStruct((_B,), jnp.float32),
    mesh=_mesh,
    compiler_params=pltpu.CompilerParams(
        needs_layout_passes=False, use_tc_tiling_on_sc=False),
    scratch_types=[
        pltpu.VMEM((_BPW,), jnp.int32),
        pltpu.VMEM((_BPW,), jnp.int32),
        pltpu.VMEM((_BPW, _F), jnp.float32),
        pltpu.VMEM((_BPW, _F), jnp.float32),
        pltpu.VMEM((_BPW,), jnp.float32),
        pltpu.SemaphoreType.DMA,
        pltpu.SemaphoreType.DMA,
    ],
)
def _sc_dot_kernel(user_h, movie_h, ut_h, mt_h, out_h,
                   uidx, midx, urows, mrows, outv, sem_u, sem_m):
    wid = lax.axis_index("s") * _NC + lax.axis_index("c")
    base = wid * _BPW

    pltpu.sync_copy(user_h.at[pl.ds(base, _BPW)], uidx)
    pltpu.sync_copy(movie_h.at[pl.ds(base, _BPW)], midx)

    cp_u = pltpu.async_copy(ut_h.at[uidx], urows, sem_u)
    cp_m = pltpu.async_copy(mt_h.at[midx], mrows, sem_m)
    cp_u.wait()
    cp_m.wait()

    lane = lax.iota(jnp.int32, 16)

    def body(i, carry):
        b0 = i * 16
        rows = lane + b0
        acc = jnp.zeros((16,), jnp.float32)
        for f in range(16):
            col = jnp.full((16,), f, jnp.int32)
            u = plsc.load_gather(urows, [rows, col])
            m = plsc.load_gather(mrows, [rows, col])
            acc = acc + u * m
        outv[pl.ds(b0, 16)] = acc
        return carry

    lax.fori_loop(0, _BPW // 16, body, 0, unroll=False)

    pltpu.sync_copy(outv, out_h.at[pl.ds(base, _BPW)])


def kernel(user, movie, user_to_feature, movie_to_feature):
    return _sc_dot_kernel(user, movie, user_to_feature, movie_to_feature)


# speedup vs baseline: 3.9216x; 3.9216x over previous
import functools

import jax
import jax.numpy as jnp
from jax import lax
from jax.experimental import pallas as pl
from jax.experimental.pallas import tpu as pltpu
from jax.experimental.pallas import tpu_sc as plsc

_B = 16384
_F = 16

_info = plsc.get_sparse_core_info()
_NC, _NS = _info.num_cores, _info.num_subcores
_NW = _NC * _NS
_BPW = _B // _NW
_MG = 128

_mesh = plsc.VectorSubcoreMesh(core_axis_name="c", subcore_axis_name="s")

_IV = tuple([0] * 8 + [1] * 8)
_SV = tuple(list(range(8)) * 2)


@functools.partial(
    pl.kernel,
    out_type=jax.ShapeDtypeStruct((_B,), jnp.float32),
    mesh=_mesh,
    compiler_params=pltpu.CompilerParams(needs_layout_passes=False),
    scratch_types=[
        pltpu.VMEM((_BPW,), jnp.int32),
        pltpu.VMEM((_BPW,), jnp.int32),
        pltpu.VMEM((_BPW,), jnp.int32),
        pltpu.VMEM((2, 16, 2, 8, 128), jnp.float32),
        pltpu.VMEM((2, 16, 128), jnp.float32),
        pltpu.VMEM((_BPW,), jnp.float32),
        pltpu.SemaphoreType.DMA,
        pltpu.SemaphoreType.DMA,
        pltpu.SemaphoreType.DMA,
        pltpu.SemaphoreType.DMA,
    ],
)
def _sc_dot_kernel(user_h, movie_h, ut3_h, mt2_h, out_h,
                   uidx, midx, mgidx, ublk, mbuf, outv,
                   sem_u0, sem_m0, sem_u1, sem_m1):
    wid = lax.axis_index("s") * _NC + lax.axis_index("c")
    base = wid * _BPW

    pltpu.sync_copy(user_h.at[pl.ds(base, _BPW)], uidx)
    pltpu.sync_copy(movie_h.at[pl.ds(base, _BPW)], midx)

    lane = lax.iota(jnp.int32, 16)
    iv = lane >> 3
    sv = lane & 7

    def mg_body(i, carry):
        mgidx[pl.ds(i * 16, 16)] = midx[pl.ds(i * 16, 16)] >> 3
        return carry

    lax.fori_loop(0, _BPW // 16, mg_body, 0, unroll=False)

    def issue_chunk(c, buf, sem_u, sem_m):
        b0 = c * 16
        u16 = uidx[pl.ds(b0, 16)]
        for r in range(16):
            j0 = pl.multiple_of((u16[r] >> 7) * 128, 128)
            pltpu.async_copy(ut3_h.at[:, :, pl.ds(j0, 128)],
                             ublk.at[buf, r], sem_u)
        pltpu.async_copy(mt2_h.at[mgidx.at[pl.ds(b0, 16)]],
                         mbuf.at[buf], sem_m)

    def wait_chunk(sem_u, sem_m):
        for _ in range(16):
            pltpu.make_async_copy(ut3_h.at[:, :, pl.ds(0, 128)],
                                  ublk.at[0, 0], sem_u).wait()
        pltpu.make_async_copy(mt2_h.at[mgidx.at[pl.ds(0, 16)]],
                              mbuf.at[0], sem_m).wait()

    def compute_chunk(c, buf):
        b0 = c * 16
        u16 = uidx[pl.ds(b0, 16)]
        m16 = midx[pl.ds(b0, 16)]
        cv = u16 & 127
        mcol0 = (m16 & 7) * 16
        bufv = jnp.zeros((16,), jnp.int32) + buf
        acc = jnp.zeros((16,), jnp.float32)
        for f in range(_F):
            hiv = jnp.full((16,), f >> 3, jnp.int32)
            lov = jnp.full((16,), f & 7, jnp.int32)
            uf = plsc.load_gather(ublk, [bufv, lane, hiv, lov, cv])
            mf = plsc.load_gather(mbuf, [bufv, lane, mcol0 + f])
            acc = acc + uf * mf
        outv[pl.ds(b0, 16)] = acc

    n_pairs = _BPW // 32
    issue_chunk(0, 0, sem_u0, sem_m0)

    def u_body(k, carry):
        c = k * 2
        issue_chunk(c + 1, 1, sem_u1, sem_m1)
        wait_chunk(sem_u0, sem_m0)
        compute_chunk(c, 0)

        @pl.when(k < n_pairs - 1)
        def _():
            issue_chunk(c + 2, 0, sem_u0, sem_m0)

        wait_chunk(sem_u1, sem_m1)
        compute_chunk(c + 1, 1)
        return carry

    lax.fori_loop(0, n_pairs, u_body, 0, unroll=False)

    pltpu.sync_copy(outv, out_h.at[pl.ds(base, _BPW)])


def kernel(user, movie, user_to_feature, movie_to_feature):
    n_u = user_to_feature.shape[0]
    n_m = movie_to_feature.shape[0]
    ut3 = user_to_feature.T.reshape(2, 8, n_u)
    mt2 = movie_to_feature.reshape(n_m // 8, 128)
    return _sc_dot_kernel(user, movie, ut3, mt2)
